# baseline (speedup 1.0000x reference)
import functools

import jax
import jax.numpy as jnp
from jax import lax
from jax.experimental import pallas as pl
from jax.experimental.pallas import tpu as pltpu
from jax.experimental.pallas import tpu_sc as plsc

S, DIM, HID, E, TOPK = 2048, 768, 3072, 8, 2
DIMW = DIM // 2
SP = S * TOPK
BLK = 128
NB = SP // BLK + E
PAD = NB * BLK
NBP = NB + 8

L = 16
NW = 32
TPW = S // NW

_SC_MESH = dict(core_axis_name="c", subcore_axis_name="s",
                num_cores=2, num_subcores=16)


def _router_body(x_ref, wr_ref, ti_ref, tw_ref):
    logits = jnp.dot(x_ref[...], wr_ref[...], preferred_element_type=jnp.float32)
    p = jax.nn.softmax(logits, axis=-1)
    col = lax.broadcasted_iota(jnp.int32, p.shape, 1)
    m1 = jnp.max(p, axis=-1, keepdims=True)
    i1 = jnp.min(jnp.where(p == m1, col, E), axis=-1, keepdims=True)
    p_rest = jnp.where(col == i1, -jnp.inf, p)
    m2 = jnp.max(p_rest, axis=-1, keepdims=True)
    i2 = jnp.min(jnp.where(p_rest == m2, col, E), axis=-1, keepdims=True)
    ti_ref[...] = jnp.concatenate([i1, i2], axis=1)
    inv = 1.0 / (m1 + m2)
    tw_ref[...] = jnp.concatenate([m1 * inv, m2 * inv], axis=1)


@jax.jit
def _router(x2d, W_router):
    return pl.pallas_call(
        _router_body,
        out_shape=[
            jax.ShapeDtypeStruct((S, TOPK), jnp.int32),
            jax.ShapeDtypeStruct((S, TOPK), jnp.float32),
        ],
    )(x2d, W_router)


NR = SP // BLK


def _bookkeep_body(tif_ref, slot_ref, eid_ref, valid_ref):
    tif = tif_ref[...]
    r = lax.broadcasted_iota(jnp.int32, (BLK, BLK), 0)
    c = lax.broadcasted_iota(jnp.int32, (BLK, BLK), 1)
    ut = (r <= c).astype(jnp.float32)
    r2 = lax.broadcasted_iota(jnp.int32, (NR, NR), 0)
    c2 = lax.broadcasted_iota(jnp.int32, (NR, NR), 1)
    lt = (r2 > c2).astype(jnp.float32)

    ranks, cnts = [], []
    for e in range(E):
        m = (tif == e).astype(jnp.float32)
        pref = jnp.dot(m, ut, preferred_element_type=jnp.float32)
        rt = pref[:, BLK - 1:BLK]
        ro = jnp.dot(lt, rt, preferred_element_type=jnp.float32)
        ranks.append((pref + ro - 1.0).astype(jnp.int32))
        cnts.append((ro[NR - 1:NR, :] + rt[NR - 1:NR, :]).astype(jnp.int32))

    offs, starts, nblks = [], [], []
    off = jnp.zeros((1, 1), jnp.int32)
    for e in range(E):
        offs.append(off)
        nb = (cnts[e] + (BLK - 1)) >> 7
        starts.append(off >> 7)
        nblks.append(nb)
        off = off + (nb << 7)

    slot = jnp.zeros((NR, BLK), jnp.int32)
    for e in range(E):
        sel = tif == e
        slot = jnp.where(sel, offs[e] + ranks[e], slot)
    slot_ref[...] = slot

    bid = lax.broadcasted_iota(jnp.int32, (1, NBP), 1)
    acc_e = jnp.zeros((1, NBP), jnp.int32)
    for e in range(1, E):
        sel = (bid >= starts[e]) & (bid < starts[e] + nblks[e])
        acc_e = jnp.where(sel, e, acc_e)
    eid_ref[...] = acc_e

    brow = lax.broadcasted_iota(jnp.int32, (NB, 1), 0)
    bcol = lax.broadcasted_iota(jnp.int32, (NB, BLK), 1)
    boff = jnp.zeros((NB, 1), jnp.int32)
    bcnt = jnp.zeros((NB, 1), jnp.int32)
    for e in range(E):
        sel = (brow >= starts[e]) & (brow < starts[e] + nblks[e])
        boff = jnp.where(sel, offs[e], boff)
        bcnt = jnp.where(sel, cnts[e], bcnt)
    p = (brow << 7) + bcol
    valid_ref[...] = ((p - boff) < bcnt).astype(jnp.int32)


@jax.jit
def _bookkeep(tif):
    return pl.pallas_call(
        _bookkeep_body,
        out_shape=[
            jax.ShapeDtypeStruct((NR, BLK), jnp.int32),
            jax.ShapeDtypeStruct((1, NBP), jnp.int32),
            jax.ShapeDtypeStruct((NB, BLK), jnp.int32),
        ],
    )(tif)


def _dispatch_body(x_hbm, sev_hbm, sod_hbm,
                   xd_hbm,
                   xbuf, sev, sod, sem):
    c = lax.axis_index("c")
    s = lax.axis_index("s")
    w = s * 2 + c
    base = pl.multiple_of(w * TPW, TPW)

    lds = [
        pltpu.async_copy(x_hbm.at[pl.ds(base, TPW)], xbuf, sem),
        pltpu.async_copy(sev_hbm.at[pl.ds(base, TPW)], sev, sem),
        pltpu.async_copy(sod_hbm.at[pl.ds(base, TPW)], sod, sem),
    ]
    for cp in lds:
        cp.wait()

    sts = [
        pltpu.async_copy(xbuf, xd_hbm.at[sev], sem),
        pltpu.async_copy(xbuf, xd_hbm.at[sod], sem),
    ]
    for cp in sts:
        cp.wait()


_dispatch = pl.kernel(
    _dispatch_body,
    out_type=jax.ShapeDtypeStruct((PAD, DIMW), jnp.int32),
    mesh=plsc.VectorSubcoreMesh(**_SC_MESH),
    scratch_types=[
        pltpu.VMEM((TPW, DIMW), jnp.int32),
        pltpu.VMEM((TPW,), jnp.int32),
        pltpu.VMEM((TPW,), jnp.int32),
        pltpu.SemaphoreType.DMA,
    ],
)


def _expert_body(eid_ref, xd_ref, wup_ref, wdn_ref, y_ref):
    xb = xd_ref[...]
    h = jnp.dot(xb, wup_ref[0], preferred_element_type=jnp.float32)
    h = jax.nn.gelu(h)
    y_ref[...] = jnp.dot(h.astype(jnp.bfloat16), wdn_ref[0],
                         preferred_element_type=jnp.float32)


@jax.jit
def _experts(eid, xd, W_up_bf, W_dn_bf):
    grid_spec = pltpu.PrefetchScalarGridSpec(
        num_scalar_prefetch=1,
        grid=(NB,),
        in_specs=[
            pl.BlockSpec((BLK, DIM), lambda g, eid_ref: (g, 0)),
            pl.BlockSpec((1, DIM, HID), lambda g, eid_ref: (eid_ref[g], 0, 0)),
            pl.BlockSpec((1, HID, DIM), lambda g, eid_ref: (eid_ref[g], 0, 0)),
        ],
        out_specs=pl.BlockSpec((BLK, DIM), lambda g, eid_ref: (g, 0)),
    )
    return pl.pallas_call(
        _expert_body,
        grid_spec=grid_spec,
        out_shape=jax.ShapeDtypeStruct((PAD, DIM), jnp.float32),
    )(eid, xd, W_up_bf, W_dn_bf)


def _combine_body(y_hbm, s0_hbm, s1_hbm, w0_hbm, w1_hbm,
                  out_hbm,
                  rows0, rows1, i0, i1, w0b, w1b, sem):
    c = lax.axis_index("c")
    s = lax.axis_index("s")
    w = s * 2 + c
    base = pl.multiple_of(w * TPW, TPW)

    lds = [
        pltpu.async_copy(s0_hbm.at[pl.ds(base, TPW)], i0, sem),
        pltpu.async_copy(s1_hbm.at[pl.ds(base, TPW)], i1, sem),
        pltpu.async_copy(w0_hbm.at[pl.ds(base, TPW)], w0b.at[pl.ds(0, TPW)], sem),
        pltpu.async_copy(w1_hbm.at[pl.ds(base, TPW)], w1b.at[pl.ds(0, TPW)], sem),
    ]
    for cp in lds:
        cp.wait()
    gts = [
        pltpu.async_copy(y_hbm.at[i0], rows0, sem),
        pltpu.async_copy(y_hbm.at[i1], rows1, sem),
    ]
    for cp in gts:
        cp.wait()

    def rbody(r, carry):
        w0 = w0b[pl.ds(r, L)][0]
        w1 = w1b[pl.ds(r, L)][0]
        for i in range(DIM // L):
            sl = pl.ds(i * L, L)
            rows0[r, sl] = rows0[r, sl] * w0 + rows1[r, sl] * w1
        return carry

    lax.fori_loop(0, TPW, rbody, jnp.int32(0))
    pltpu.sync_copy(rows0, out_hbm.at[pl.ds(base, TPW)])


_combine = pl.kernel(
    _combine_body,
    out_type=jax.ShapeDtypeStruct((S, DIM), jnp.float32),
    mesh=plsc.VectorSubcoreMesh(**_SC_MESH),
    scratch_types=[
        pltpu.VMEM((TPW, DIM), jnp.float32),
        pltpu.VMEM((TPW, DIM), jnp.float32),
        pltpu.VMEM((TPW,), jnp.int32),
        pltpu.VMEM((TPW,), jnp.int32),
        pltpu.VMEM((TPW + L,), jnp.float32),
        pltpu.VMEM((TPW + L,), jnp.float32),
        pltpu.SemaphoreType.DMA,
    ],
)


def kernel(x, W_router, W_up, W_down):
    x2d = x.reshape(S, DIM)
    ti, tw = _router(x2d, W_router)

    tif = ti.T.reshape(NR, BLK)
    twf = tw.T.reshape(TOPK, S)

    slot, eid, valid = _bookkeep(tif)
    del valid
    slot2 = slot.reshape(TOPK, S)

    xp = lax.bitcast_convert_type(
        x2d.astype(jnp.bfloat16).reshape(S, DIMW, 2), jnp.int32
    )
    xd = _dispatch(xp, slot2[0], slot2[1])
    xd_bf = lax.bitcast_convert_type(xd, jnp.bfloat16).reshape(PAD, DIM)

    y = _experts(eid.reshape(NBP)[:NB], xd_bf,
                 W_up.astype(jnp.bfloat16), W_down.astype(jnp.bfloat16))

    out = _combine(y, slot2[0], slot2[1], twf[0], twf[1])
    return out.reshape(x.shape)


# speedup vs baseline: 1.5420x; 1.5420x over previous
import functools

import jax
import jax.numpy as jnp
from jax import lax
from jax.experimental import pallas as pl
from jax.experimental.pallas import tpu as pltpu
from jax.experimental.pallas import tpu_sc as plsc

S, DIM, HID, E, TOPK = 2048, 768, 3072, 8, 2
DIMW = DIM // 2
SP = S * TOPK
BLK = 128
NB = SP // BLK + E
PAD = NB * BLK
NBP = NB + 8

L = 16
NW = 32
TPW = S // NW

_SC_MESH = dict(core_axis_name="c", subcore_axis_name="s",
                num_cores=2, num_subcores=16)


def _router_body(x_ref, wr_ref, ti_ref, tw_ref):
    logits = jnp.dot(x_ref[...], wr_ref[...], preferred_element_type=jnp.float32)
    p = jax.nn.softmax(logits, axis=-1)
    col = lax.broadcasted_iota(jnp.int32, p.shape, 1)
    m1 = jnp.max(p, axis=-1, keepdims=True)
    i1 = jnp.min(jnp.where(p == m1, col, E), axis=-1, keepdims=True)
    p_rest = jnp.where(col == i1, -jnp.inf, p)
    m2 = jnp.max(p_rest, axis=-1, keepdims=True)
    i2 = jnp.min(jnp.where(p_rest == m2, col, E), axis=-1, keepdims=True)
    ti_ref[...] = jnp.concatenate([i1, i2], axis=1)
    inv = 1.0 / (m1 + m2)
    tw_ref[...] = jnp.concatenate([m1 * inv, m2 * inv], axis=1)


@jax.jit
def _router(x2d, W_router):
    return pl.pallas_call(
        _router_body,
        out_shape=[
            jax.ShapeDtypeStruct((S, TOPK), jnp.int32),
            jax.ShapeDtypeStruct((S, TOPK), jnp.float32),
        ],
    )(x2d, W_router)


NR = SP // BLK


def _bookkeep_body(tif_ref, slot_ref, eid_ref, valid_ref):
    tif = tif_ref[...]
    r = lax.broadcasted_iota(jnp.int32, (BLK, BLK), 0)
    c = lax.broadcasted_iota(jnp.int32, (BLK, BLK), 1)
    ut = (r <= c).astype(jnp.float32)
    r2 = lax.broadcasted_iota(jnp.int32, (NR, NR), 0)
    c2 = lax.broadcasted_iota(jnp.int32, (NR, NR), 1)
    lt = (r2 > c2).astype(jnp.float32)

    ranks, cnts = [], []
    for e in range(E):
        m = (tif == e).astype(jnp.float32)
        pref = jnp.dot(m, ut, preferred_element_type=jnp.float32)
        rt = pref[:, BLK - 1:BLK]
        ro = jnp.dot(lt, rt, preferred_element_type=jnp.float32)
        ranks.append((pref + ro - 1.0).astype(jnp.int32))
        cnts.append((ro[NR - 1:NR, :] + rt[NR - 1:NR, :]).astype(jnp.int32))

    offs, starts, nblks = [], [], []
    off = jnp.zeros((1, 1), jnp.int32)
    for e in range(E):
        offs.append(off)
        nb = (cnts[e] + (BLK - 1)) >> 7
        starts.append(off >> 7)
        nblks.append(nb)
        off = off + (nb << 7)

    slot = jnp.zeros((NR, BLK), jnp.int32)
    for e in range(E):
        sel = tif == e
        slot = jnp.where(sel, offs[e] + ranks[e], slot)
    slot_ref[...] = slot

    bid = lax.broadcasted_iota(jnp.int32, (1, NBP), 1)
    acc_e = jnp.zeros((1, NBP), jnp.int32)
    for e in range(1, E):
        sel = (bid >= starts[e]) & (bid < starts[e] + nblks[e])
        acc_e = jnp.where(sel, e, acc_e)
    eid_ref[...] = acc_e

    brow = lax.broadcasted_iota(jnp.int32, (NB, 1), 0)
    bcol = lax.broadcasted_iota(jnp.int32, (NB, BLK), 1)
    boff = jnp.zeros((NB, 1), jnp.int32)
    bcnt = jnp.zeros((NB, 1), jnp.int32)
    for e in range(E):
        sel = (brow >= starts[e]) & (brow < starts[e] + nblks[e])
        boff = jnp.where(sel, offs[e], boff)
        bcnt = jnp.where(sel, cnts[e], bcnt)
    p = (brow << 7) + bcol
    valid_ref[...] = ((p - boff) < bcnt).astype(jnp.int32)


@jax.jit
def _bookkeep(tif):
    return pl.pallas_call(
        _bookkeep_body,
        out_shape=[
            jax.ShapeDtypeStruct((NR, BLK), jnp.int32),
            jax.ShapeDtypeStruct((1, NBP), jnp.int32),
            jax.ShapeDtypeStruct((NB, BLK), jnp.int32),
        ],
    )(tif)


def _dispatch_body(x_hbm, sev_hbm, sod_hbm,
                   xd_hbm,
                   xbuf, sev, sod, sem):
    c = lax.axis_index("c")
    s = lax.axis_index("s")
    w = s * 2 + c
    base = pl.multiple_of(w * TPW, TPW)

    lds = [
        pltpu.async_copy(x_hbm.at[pl.ds(base, TPW)], xbuf, sem),
        pltpu.async_copy(sev_hbm.at[pl.ds(base, TPW)], sev, sem),
        pltpu.async_copy(sod_hbm.at[pl.ds(base, TPW)], sod, sem),
    ]
    for cp in lds:
        cp.wait()

    sts = [
        pltpu.async_copy(xbuf, xd_hbm.at[sev], sem),
        pltpu.async_copy(xbuf, xd_hbm.at[sod], sem),
    ]
    for cp in sts:
        cp.wait()


_dispatch = pl.kernel(
    _dispatch_body,
    out_type=jax.ShapeDtypeStruct((PAD, DIM), jnp.float32),
    mesh=plsc.VectorSubcoreMesh(**_SC_MESH),
    scratch_types=[
        pltpu.VMEM((TPW, DIM), jnp.float32),
        pltpu.VMEM((TPW,), jnp.int32),
        pltpu.VMEM((TPW,), jnp.int32),
        pltpu.SemaphoreType.DMA,
    ],
)


def _expert_body(eid_ref, xd_ref, wup_ref, wdn_ref, y_ref):
    xb = xd_ref[...].astype(jnp.bfloat16)
    h = jnp.dot(xb, wup_ref[0], preferred_element_type=jnp.float32)
    h = jax.nn.gelu(h)
    y_ref[...] = jnp.dot(h.astype(jnp.bfloat16), wdn_ref[0],
                         preferred_element_type=jnp.float32)


@jax.jit
def _experts(eid, xd, W_up_bf, W_dn_bf):
    grid_spec = pltpu.PrefetchScalarGridSpec(
        num_scalar_prefetch=1,
        grid=(NB,),
        in_specs=[
            pl.BlockSpec((BLK, DIM), lambda g, eid_ref: (g, 0)),
            pl.BlockSpec((1, DIM, HID), lambda g, eid_ref: (eid_ref[g], 0, 0)),
            pl.BlockSpec((1, HID, DIM), lambda g, eid_ref: (eid_ref[g], 0, 0)),
        ],
        out_specs=pl.BlockSpec((BLK, DIM), lambda g, eid_ref: (g, 0)),
    )
    return pl.pallas_call(
        _expert_body,
        grid_spec=grid_spec,
        out_shape=jax.ShapeDtypeStruct((PAD, DIM), jnp.float32),
    )(eid, xd, W_up_bf, W_dn_bf)


def _combine_body(y_hbm, s0_hbm, s1_hbm, w0_hbm, w1_hbm,
                  out_hbm,
                  rows0, rows1, i0, i1, w0b, w1b, sem):
    c = lax.axis_index("c")
    s = lax.axis_index("s")
    w = s * 2 + c
    base = pl.multiple_of(w * TPW, TPW)

    lds = [
        pltpu.async_copy(s0_hbm.at[pl.ds(base, TPW)], i0, sem),
        pltpu.async_copy(s1_hbm.at[pl.ds(base, TPW)], i1, sem),
        pltpu.async_copy(w0_hbm.at[pl.ds(base, TPW)], w0b.at[pl.ds(0, TPW)], sem),
        pltpu.async_copy(w1_hbm.at[pl.ds(base, TPW)], w1b.at[pl.ds(0, TPW)], sem),
    ]
    for cp in lds:
        cp.wait()
    gts = [
        pltpu.async_copy(y_hbm.at[i0], rows0, sem),
        pltpu.async_copy(y_hbm.at[i1], rows1, sem),
    ]
    for cp in gts:
        cp.wait()

    def rbody(r, carry):
        w0 = w0b[pl.ds(r, L)][0]
        w1 = w1b[pl.ds(r, L)][0]
        for i in range(DIM // L):
            sl = pl.ds(i * L, L)
            rows0[r, sl] = rows0[r, sl] * w0 + rows1[r, sl] * w1
        return carry

    lax.fori_loop(0, TPW, rbody, jnp.int32(0))
    pltpu.sync_copy(rows0, out_hbm.at[pl.ds(base, TPW)])


_combine = pl.kernel(
    _combine_body,
    out_type=jax.ShapeDtypeStruct((S, DIM), jnp.float32),
    mesh=plsc.VectorSubcoreMesh(**_SC_MESH),
    scratch_types=[
        pltpu.VMEM((TPW, DIM), jnp.float32),
        pltpu.VMEM((TPW, DIM), jnp.float32),
        pltpu.VMEM((TPW,), jnp.int32),
        pltpu.VMEM((TPW,), jnp.int32),
        pltpu.VMEM((TPW + L,), jnp.float32),
        pltpu.VMEM((TPW + L,), jnp.float32),
        pltpu.SemaphoreType.DMA,
    ],
)


def kernel(x, W_router, W_up, W_down):
    x2d = x.reshape(S, DIM)
    ti, tw = _router(x2d, W_router)

    tif = ti.T.reshape(NR, BLK)
    twf = tw.T.reshape(TOPK, S)

    slot, eid, valid = _bookkeep(tif)
    del valid
    slot2 = slot.reshape(TOPK, S)

    xd = _dispatch(x2d, slot2[0], slot2[1])

    y = _experts(eid.reshape(NBP)[:NB], xd,
                 W_up.astype(jnp.bfloat16), W_down.astype(jnp.bfloat16))

    out = _combine(y, slot2[0], slot2[1], twf[0], twf[1])
    return out.reshape(x.shape)


# speedup vs baseline: 1.8567x; 1.2041x over previous
import functools

import jax
import jax.numpy as jnp
from jax import lax
from jax.experimental import pallas as pl
from jax.experimental.pallas import tpu as pltpu
from jax.experimental.pallas import tpu_sc as plsc

S, DIM, HID, E, TOPK = 2048, 768, 3072, 8, 2
DIMW = DIM // 2
SP = S * TOPK
BLK = 128
NB = SP // BLK + E
PAD = NB * BLK
NBP = NB + 8

L = 16
NW = 32
TPW = S // NW

_SC_MESH = dict(core_axis_name="c", subcore_axis_name="s",
                num_cores=2, num_subcores=16)


def _router_body(x_ref, wr_ref, ti_ref, tw_ref):
    logits = jnp.dot(x_ref[...], wr_ref[...], preferred_element_type=jnp.float32)
    p = jax.nn.softmax(logits, axis=-1)
    col = lax.broadcasted_iota(jnp.int32, p.shape, 1)
    m1 = jnp.max(p, axis=-1, keepdims=True)
    i1 = jnp.min(jnp.where(p == m1, col, E), axis=-1, keepdims=True)
    p_rest = jnp.where(col == i1, -jnp.inf, p)
    m2 = jnp.max(p_rest, axis=-1, keepdims=True)
    i2 = jnp.min(jnp.where(p_rest == m2, col, E), axis=-1, keepdims=True)
    ti_ref[...] = jnp.concatenate([i1, i2], axis=1)
    inv = 1.0 / (m1 + m2)
    tw_ref[...] = jnp.concatenate([m1 * inv, m2 * inv], axis=1)


@jax.jit
def _router(x2d, W_router):
    return pl.pallas_call(
        _router_body,
        out_shape=[
            jax.ShapeDtypeStruct((S, TOPK), jnp.int32),
            jax.ShapeDtypeStruct((S, TOPK), jnp.float32),
        ],
    )(x2d, W_router)


NR = SP // BLK


def _bookkeep_body(tif_ref, slot_ref, eid_ref, valid_ref):
    tif = tif_ref[...]
    r = lax.broadcasted_iota(jnp.int32, (BLK, BLK), 0)
    c = lax.broadcasted_iota(jnp.int32, (BLK, BLK), 1)
    ut = (r <= c).astype(jnp.float32)
    r2 = lax.broadcasted_iota(jnp.int32, (NR, NR), 0)
    c2 = lax.broadcasted_iota(jnp.int32, (NR, NR), 1)
    lt = (r2 > c2).astype(jnp.float32)

    ranks, cnts = [], []
    for e in range(E):
        m = (tif == e).astype(jnp.float32)
        pref = jnp.dot(m, ut, preferred_element_type=jnp.float32)
        rt = pref[:, BLK - 1:BLK]
        ro = jnp.dot(lt, rt, preferred_element_type=jnp.float32)
        ranks.append((pref + ro - 1.0).astype(jnp.int32))
        cnts.append((ro[NR - 1:NR, :] + rt[NR - 1:NR, :]).astype(jnp.int32))

    offs, starts, nblks = [], [], []
    off = jnp.zeros((1, 1), jnp.int32)
    for e in range(E):
        offs.append(off)
        nb = (cnts[e] + (BLK - 1)) >> 7
        starts.append(off >> 7)
        nblks.append(nb)
        off = off + (nb << 7)

    slot = jnp.zeros((NR, BLK), jnp.int32)
    for e in range(E):
        sel = tif == e
        slot = jnp.where(sel, offs[e] + ranks[e], slot)
    slot_ref[...] = slot

    bid = lax.broadcasted_iota(jnp.int32, (1, NBP), 1)
    acc_e = jnp.zeros((1, NBP), jnp.int32)
    for e in range(1, E):
        sel = (bid >= starts[e]) & (bid < starts[e] + nblks[e])
        acc_e = jnp.where(sel, e, acc_e)
    eid_ref[...] = acc_e

    brow = lax.broadcasted_iota(jnp.int32, (NB, 1), 0)
    bcol = lax.broadcasted_iota(jnp.int32, (NB, BLK), 1)
    boff = jnp.zeros((NB, 1), jnp.int32)
    bcnt = jnp.zeros((NB, 1), jnp.int32)
    for e in range(E):
        sel = (brow >= starts[e]) & (brow < starts[e] + nblks[e])
        boff = jnp.where(sel, offs[e], boff)
        bcnt = jnp.where(sel, cnts[e], bcnt)
    p = (brow << 7) + bcol
    valid_ref[...] = ((p - boff) < bcnt).astype(jnp.int32)


@jax.jit
def _bookkeep(tif):
    return pl.pallas_call(
        _bookkeep_body,
        out_shape=[
            jax.ShapeDtypeStruct((NR, BLK), jnp.int32),
            jax.ShapeDtypeStruct((1, NBP), jnp.int32),
            jax.ShapeDtypeStruct((NB, BLK), jnp.int32),
        ],
    )(tif)


def _dispatch_body(x_hbm, sev_hbm, sod_hbm,
                   xd_hbm,
                   xbuf, sev, sod, sem):
    c = lax.axis_index("c")
    s = lax.axis_index("s")
    w = s * 2 + c
    base = pl.multiple_of(w * TPW, TPW)

    lds = [
        pltpu.async_copy(x_hbm.at[pl.ds(base, TPW)], xbuf, sem),
        pltpu.async_copy(sev_hbm.at[pl.ds(base, TPW)], sev, sem),
        pltpu.async_copy(sod_hbm.at[pl.ds(base, TPW)], sod, sem),
    ]
    for cp in lds:
        cp.wait()

    sts = [
        pltpu.async_copy(xbuf, xd_hbm.at[sev], sem),
        pltpu.async_copy(xbuf, xd_hbm.at[sod], sem),
    ]
    for cp in sts:
        cp.wait()


_dispatch = pl.kernel(
    _dispatch_body,
    out_type=jax.ShapeDtypeStruct((PAD, DIM), jnp.float32),
    mesh=plsc.VectorSubcoreMesh(**_SC_MESH),
    scratch_types=[
        pltpu.VMEM((TPW, DIM), jnp.float32),
        pltpu.VMEM((TPW,), jnp.int32),
        pltpu.VMEM((TPW,), jnp.int32),
        pltpu.SemaphoreType.DMA,
    ],
)


def _expert_body(eid_ref, xd_ref, wup_ref, wdn_ref, y_ref, wub, wdb):
    g = pl.program_id(0)
    prev = eid_ref[jnp.maximum(g - 1, 0)]

    @pl.when((g == 0) | (eid_ref[g] != prev))
    def _cast_weights():
        wub[...] = wup_ref[0].astype(jnp.bfloat16)
        wdb[...] = wdn_ref[0].astype(jnp.bfloat16)

    xb = xd_ref[...].astype(jnp.bfloat16)
    h = jnp.dot(xb, wub[...], preferred_element_type=jnp.float32)
    h = jax.nn.gelu(h)
    y_ref[...] = jnp.dot(h.astype(jnp.bfloat16), wdb[...],
                         preferred_element_type=jnp.float32)


@jax.jit
def _experts(eid, xd, W_up, W_down):
    grid_spec = pltpu.PrefetchScalarGridSpec(
        num_scalar_prefetch=1,
        grid=(NB,),
        in_specs=[
            pl.BlockSpec((BLK, DIM), lambda g, eid_ref: (g, 0)),
            pl.BlockSpec((1, DIM, HID), lambda g, eid_ref: (eid_ref[g], 0, 0)),
            pl.BlockSpec((1, HID, DIM), lambda g, eid_ref: (eid_ref[g], 0, 0)),
        ],
        out_specs=pl.BlockSpec((BLK, DIM), lambda g, eid_ref: (g, 0)),
        scratch_shapes=[
            pltpu.VMEM((DIM, HID), jnp.bfloat16),
            pltpu.VMEM((HID, DIM), jnp.bfloat16),
        ],
    )
    return pl.pallas_call(
        _expert_body,
        grid_spec=grid_spec,
        out_shape=jax.ShapeDtypeStruct((PAD, DIM), jnp.float32),
    )(eid, xd, W_up, W_down)


def _combine_body(y_hbm, s0_hbm, s1_hbm, w0_hbm, w1_hbm,
                  out_hbm,
                  rows0, rows1, i0, i1, w0b, w1b, sem):
    c = lax.axis_index("c")
    s = lax.axis_index("s")
    w = s * 2 + c
    base = pl.multiple_of(w * TPW, TPW)

    lds = [
        pltpu.async_copy(s0_hbm.at[pl.ds(base, TPW)], i0, sem),
        pltpu.async_copy(s1_hbm.at[pl.ds(base, TPW)], i1, sem),
        pltpu.async_copy(w0_hbm.at[pl.ds(base, TPW)], w0b.at[pl.ds(0, TPW)], sem),
        pltpu.async_copy(w1_hbm.at[pl.ds(base, TPW)], w1b.at[pl.ds(0, TPW)], sem),
    ]
    for cp in lds:
        cp.wait()
    gts = [
        pltpu.async_copy(y_hbm.at[i0], rows0, sem),
        pltpu.async_copy(y_hbm.at[i1], rows1, sem),
    ]
    for cp in gts:
        cp.wait()

    def rbody(r, carry):
        w0 = w0b[pl.ds(r, L)][0]
        w1 = w1b[pl.ds(r, L)][0]
        for i in range(DIM // L):
            sl = pl.ds(i * L, L)
            rows0[r, sl] = rows0[r, sl] * w0 + rows1[r, sl] * w1
        return carry

    lax.fori_loop(0, TPW, rbody, jnp.int32(0))
    pltpu.sync_copy(rows0, out_hbm.at[pl.ds(base, TPW)])


_combine = pl.kernel(
    _combine_body,
    out_type=jax.ShapeDtypeStruct((S, DIM), jnp.float32),
    mesh=plsc.VectorSubcoreMesh(**_SC_MESH),
    scratch_types=[
        pltpu.VMEM((TPW, DIM), jnp.float32),
        pltpu.VMEM((TPW, DIM), jnp.float32),
        pltpu.VMEM((TPW,), jnp.int32),
        pltpu.VMEM((TPW,), jnp.int32),
        pltpu.VMEM((TPW + L,), jnp.float32),
        pltpu.VMEM((TPW + L,), jnp.float32),
        pltpu.SemaphoreType.DMA,
    ],
)


def kernel(x, W_router, W_up, W_down):
    x2d = x.reshape(S, DIM)
    ti, tw = _router(x2d, W_router)

    tif = ti.T.reshape(NR, BLK)
    twf = tw.T.reshape(TOPK, S)

    slot, eid, valid = _bookkeep(tif)
    del valid
    slot2 = slot.reshape(TOPK, S)

    xd = _dispatch(x2d, slot2[0], slot2[1])

    y = _experts(eid.reshape(NBP)[:NB], xd, W_up, W_down)

    out = _combine(y, slot2[0], slot2[1], twf[0], twf[1])
    return out.reshape(x.shape)
